# baseline (speedup 1.0000x reference)
import jax
import jax.numpy as jnp
from jax import lax
from jax.experimental import pallas as pl
from jax.experimental.pallas import tpu as pltpu
from jax.experimental.pallas import tpu_sc as plsc

_NUM_ITEM = 1000000
_EMB = 32
_NEG = 4
_OUT_D = 2 * _EMB + 1

_NC = 2
_NS = 16
_NW = _NC * _NS

_DMA_ROWS = 128
_CHUNK = 512
_DMAS_PER_CHUNK = _CHUNK // _DMA_ROWS


def _build_sc_call(total_rows):
    rows_per_w = total_rows // _NW
    n_chunks = rows_per_w // _CHUNK
    idx_rows = rows_per_w // _DMA_ROWS

    mesh = plsc.VectorSubcoreMesh(
        core_axis_name="c", subcore_axis_name="s",
        num_cores=_NC, num_subcores=_NS)

    def body(nu_hbm, ni_hbm, ue_hbm, ie_hbm, out_hbm,
             idx_u, idx_i, u_v, i_v, out_v, sem):
        wid = lax.axis_index("s") * _NC + lax.axis_index("c")
        pltpu.sync_copy(nu_hbm.at[pl.ds(wid * rows_per_w, rows_per_w)], idx_u)
        pltpu.sync_copy(ni_hbm.at[pl.ds(wid * rows_per_w, rows_per_w)], idx_i)

        for c in range(n_chunks):
            base = wid * rows_per_w + c * _CHUNK
            copies = []
            for b in range(_DMAS_PER_CHUNK):
                j = (c * _DMAS_PER_CHUNK + b) * _DMA_ROWS
                copies.append(pltpu.async_copy(
                    ue_hbm.at[idx_u.at[pl.ds(j, _DMA_ROWS)]],
                    u_v.at[pl.ds(b * _DMA_ROWS, _DMA_ROWS)], sem))
                copies.append(pltpu.async_copy(
                    ie_hbm.at[idx_i.at[pl.ds(j, _DMA_ROWS)]],
                    i_v.at[pl.ds(b * _DMA_ROWS, _DMA_ROWS)], sem))
            for cp in copies:
                cp.wait()

            col64 = jnp.full((16,), 2 * _EMB, jnp.int32)
            lanes = lax.iota(jnp.int32, 16)

            def grp(g, _):
                r0 = g * 16
                rows = r0 + lanes
                acc = jnp.zeros((16,), jnp.float32)
                for cc in range(_EMB):
                    colv = jnp.full((16,), cc, jnp.int32)
                    acc = acc + (plsc.load_gather(u_v, [rows, colv]) *
                                 plsc.load_gather(i_v, [rows, colv]))
                plsc.store_scatter(out_v, [rows, col64], acc)
                for k in range(16):
                    r = r0 + k
                    out_v[r, pl.ds(0, 16)] = u_v[r, pl.ds(0, 16)]
                    out_v[r, pl.ds(16, 16)] = u_v[r, pl.ds(16, 16)]
                    out_v[r, pl.ds(32, 16)] = i_v[r, pl.ds(0, 16)]
                    out_v[r, pl.ds(48, 16)] = i_v[r, pl.ds(16, 16)]
                return 0

            lax.fori_loop(0, _CHUNK // 16, grp, 0)
            pltpu.sync_copy(out_v, out_hbm.at[pl.ds(base, _CHUNK)])

    return pl.kernel(
        body,
        out_type=jax.ShapeDtypeStruct((total_rows, _OUT_D), jnp.float32),
        mesh=mesh,
        compiler_params=pltpu.CompilerParams(
            needs_layout_passes=False, use_tc_tiling_on_sc=False),
        scratch_types=[
            pltpu.VMEM((rows_per_w,), jnp.int32),
            pltpu.VMEM((rows_per_w,), jnp.int32),
            pltpu.VMEM((_CHUNK, _EMB), jnp.float32),
            pltpu.VMEM((_CHUNK, _EMB), jnp.float32),
            pltpu.VMEM((_CHUNK, _OUT_D), jnp.float32),
            pltpu.SemaphoreType.DMA,
        ],
    )


def kernel(user, item, user_emb, item_emb):
    B = user.shape[0]
    total = B * (1 + _NEG)
    neg_item = jax.random.randint(
        jax.random.key(42), (B * _NEG,), 0, _NUM_ITEM, dtype=jnp.int32)
    new_user = jnp.concatenate([user, jnp.repeat(user, _NEG)], axis=0)
    new_item = jnp.concatenate([item, neg_item], axis=0)

    call = _build_sc_call(total)
    return call(new_user, new_item, user_emb, item_emb)


# speedup vs baseline: 1.0298x; 1.0298x over previous
import jax
import jax.numpy as jnp
from jax import lax
from jax.experimental import pallas as pl
from jax.experimental.pallas import tpu as pltpu
from jax.experimental.pallas import tpu_sc as plsc

_NUM_ITEM = 1000000
_EMB = 32
_NEG = 4
_OUT_D = 2 * _EMB + 1

_NC = 2
_NS = 16
_NW = _NC * _NS

_DMA_ROWS = 128
_CHUNK = 512
_DMAS_PER_CHUNK = _CHUNK // _DMA_ROWS


def _build_sc_call(total_rows):
    rows_per_w = total_rows // _NW
    n_chunks = rows_per_w // _CHUNK
    idx_rows = rows_per_w // _DMA_ROWS

    mesh = plsc.VectorSubcoreMesh(
        core_axis_name="c", subcore_axis_name="s",
        num_cores=_NC, num_subcores=_NS)

    def body(nu_hbm, ni_hbm, ue_hbm, ie_hbm, out_hbm,
             idx_u, idx_i, u_v, i_v, out_v, sem):
        wid = lax.axis_index("s") * _NC + lax.axis_index("c")
        pltpu.sync_copy(nu_hbm.at[pl.ds(wid * rows_per_w, rows_per_w)], idx_u)
        pltpu.sync_copy(ni_hbm.at[pl.ds(wid * rows_per_w, rows_per_w)], idx_i)

        for c in range(n_chunks):
            base = wid * rows_per_w + c * _CHUNK
            copies = []
            for b in range(_DMAS_PER_CHUNK):
                j = (c * _DMAS_PER_CHUNK + b) * _DMA_ROWS
                copies.append(pltpu.async_copy(
                    ue_hbm.at[idx_u.at[pl.ds(j, _DMA_ROWS)]],
                    u_v.at[pl.ds(b * _DMA_ROWS, _DMA_ROWS)], sem))
                copies.append(pltpu.async_copy(
                    ie_hbm.at[idx_i.at[pl.ds(j, _DMA_ROWS)]],
                    i_v.at[pl.ds(b * _DMA_ROWS, _DMA_ROWS)], sem))
            for cp in copies:
                cp.wait()

            lanes = lax.iota(jnp.int32, 16)
            zeros16 = jnp.zeros((16,), jnp.int32)

            def grp(g, _):
                r0 = g * 16
                rows = r0 + lanes
                acc = jnp.zeros((16,), jnp.float32)
                for cc in range(_EMB):
                    colv = jnp.full((16,), cc, jnp.int32)
                    acc = acc + (plsc.load_gather(u_v, [rows, colv]) *
                                 plsc.load_gather(i_v, [rows, colv]))
                plsc.store_scatter(out_v, [rows, zeros16], acc)
                return 0

            lax.fori_loop(0, _CHUNK // 16, grp, 0)
            pltpu.sync_copy(u_v, out_hbm.at[pl.ds(base, _CHUNK), pl.ds(0, _EMB)])
            pltpu.sync_copy(i_v, out_hbm.at[pl.ds(base, _CHUNK), pl.ds(_EMB, _EMB)])
            pltpu.sync_copy(out_v, out_hbm.at[pl.ds(base, _CHUNK), pl.ds(2 * _EMB, 1)])

    return pl.kernel(
        body,
        out_type=jax.ShapeDtypeStruct((total_rows, _OUT_D), jnp.float32),
        mesh=mesh,
        compiler_params=pltpu.CompilerParams(
            needs_layout_passes=False, use_tc_tiling_on_sc=False),
        scratch_types=[
            pltpu.VMEM((rows_per_w,), jnp.int32),
            pltpu.VMEM((rows_per_w,), jnp.int32),
            pltpu.VMEM((_CHUNK, _EMB), jnp.float32),
            pltpu.VMEM((_CHUNK, _EMB), jnp.float32),
            pltpu.VMEM((_CHUNK, 1), jnp.float32),
            pltpu.SemaphoreType.DMA,
        ],
    )


def kernel(user, item, user_emb, item_emb):
    B = user.shape[0]
    total = B * (1 + _NEG)
    neg_item = jax.random.randint(
        jax.random.key(42), (B * _NEG,), 0, _NUM_ITEM, dtype=jnp.int32)
    new_user = jnp.concatenate([user, jnp.repeat(user, _NEG)], axis=0)
    new_item = jnp.concatenate([item, neg_item], axis=0)

    call = _build_sc_call(total)
    return call(new_user, new_item, user_emb, item_emb)


# speedup vs baseline: 1.0418x; 1.0117x over previous
import jax
import jax.numpy as jnp
from jax import lax
from jax.experimental import pallas as pl
from jax.experimental.pallas import tpu as pltpu
from jax.experimental.pallas import tpu_sc as plsc

_NUM_ITEM = 1000000
_EMB = 32
_NEG = 4
_OUT_D = 2 * _EMB + 1

_NC = 2
_NS = 16
_NW = _NC * _NS

_DMA_ROWS = 128
_CHUNK = 512
_DPC = _CHUNK // _DMA_ROWS
_NBUF = 3


def _build_sc_call(total_rows):
    rows_per_w = total_rows // _NW
    n_chunks = rows_per_w // _CHUNK

    mesh = plsc.VectorSubcoreMesh(
        core_axis_name="c", subcore_axis_name="s",
        num_cores=_NC, num_subcores=_NS)

    def body(nu_hbm, ni_hbm, ue_hbm, ie_hbm, out_hbm, idx_u, idx_i, *bufs):
        u_v = bufs[0:_NBUF]
        i_v = bufs[_NBUF:2 * _NBUF]
        d_v = bufs[2 * _NBUF:3 * _NBUF]
        gsem = bufs[3 * _NBUF:4 * _NBUF]
        wsem = bufs[4 * _NBUF:5 * _NBUF]

        wid = lax.axis_index("s") * _NC + lax.axis_index("c")
        pltpu.sync_copy(nu_hbm.at[pl.ds(wid * rows_per_w, rows_per_w)], idx_u)
        pltpu.sync_copy(ni_hbm.at[pl.ds(wid * rows_per_w, rows_per_w)], idx_i)

        def fire(c):
            b = c % _NBUF
            cps = []
            for k in range(_DPC):
                off = c * _CHUNK + k * _DMA_ROWS
                dst = pl.ds(k * _DMA_ROWS, _DMA_ROWS)
                cps.append(pltpu.async_copy(
                    ue_hbm.at[idx_u.at[pl.ds(off, _DMA_ROWS)]],
                    u_v[b].at[dst], gsem[b]))
                cps.append(pltpu.async_copy(
                    ie_hbm.at[idx_i.at[pl.ds(off, _DMA_ROWS)]],
                    i_v[b].at[dst], gsem[b]))
            return cps

        lanes = lax.iota(jnp.int32, 16)
        zeros16 = jnp.zeros((16,), jnp.int32)

        gath = {0: fire(0), 1: fire(1)}
        writes = {}
        for c in range(n_chunks):
            b = c % _NBUF
            for g in gath.pop(c):
                g.wait()
            if c + 2 < n_chunks:
                if c - 1 >= 0:
                    for w in writes.pop(c - 1):
                        w.wait()
                gath[c + 2] = fire(c + 2)

            def grp(g, _, ub=u_v[b], ib=i_v[b], db=d_v[b]):
                rows = g * 16 + lanes
                acc = jnp.zeros((16,), jnp.float32)
                for cc in range(_EMB):
                    colv = jnp.full((16,), cc, jnp.int32)
                    acc = acc + (plsc.load_gather(ub, [rows, colv]) *
                                 plsc.load_gather(ib, [rows, colv]))
                plsc.store_scatter(db, [rows, zeros16], acc)
                return 0

            lax.fori_loop(0, _CHUNK // 16, grp, 0)

            base = wid * rows_per_w + c * _CHUNK
            rows_sl = pl.ds(base, _CHUNK)
            writes[c] = [
                pltpu.async_copy(
                    u_v[b], out_hbm.at[rows_sl, pl.ds(0, _EMB)], wsem[b]),
                pltpu.async_copy(
                    i_v[b], out_hbm.at[rows_sl, pl.ds(_EMB, _EMB)], wsem[b]),
                pltpu.async_copy(
                    d_v[b], out_hbm.at[rows_sl, pl.ds(2 * _EMB, 1)], wsem[b]),
            ]
        for c in sorted(writes):
            for w in writes[c]:
                w.wait()

    scratch = (
        [pltpu.VMEM((_CHUNK, _EMB), jnp.float32) for _ in range(_NBUF)] +
        [pltpu.VMEM((_CHUNK, _EMB), jnp.float32) for _ in range(_NBUF)] +
        [pltpu.VMEM((_CHUNK, 1), jnp.float32) for _ in range(_NBUF)] +
        [pltpu.SemaphoreType.DMA for _ in range(2 * _NBUF)]
    )

    return pl.kernel(
        body,
        out_type=jax.ShapeDtypeStruct((total_rows, _OUT_D), jnp.float32),
        mesh=mesh,
        compiler_params=pltpu.CompilerParams(
            needs_layout_passes=False, use_tc_tiling_on_sc=False),
        scratch_types=[
            pltpu.VMEM((rows_per_w,), jnp.int32),
            pltpu.VMEM((rows_per_w,), jnp.int32),
        ] + scratch,
    )


def kernel(user, item, user_emb, item_emb):
    B = user.shape[0]
    total = B * (1 + _NEG)
    neg_item = jax.random.randint(
        jax.random.key(42), (B * _NEG,), 0, _NUM_ITEM, dtype=jnp.int32)
    new_user = jnp.concatenate([user, jnp.repeat(user, _NEG)], axis=0)
    new_item = jnp.concatenate([item, neg_item], axis=0)

    call = _build_sc_call(total)
    return call(new_user, new_item, user_emb, item_emb)
